# initial kernel scaffold (unmeasured)
import jax
import jax.numpy as jnp
from jax import lax
from jax.experimental import pallas as pl
from jax.experimental.pallas import tpu as pltpu

N_DEV = 4
SQ = 2048
D_MODEL = 1024
HQ_LOCAL = 8
DH = 128
WINDOW = 128
SCALE = 0.08838834764831843
QBLK = 512
N_QBLK = SQ // QBLK
D_LOCAL = HQ_LOCAL * DH


def _key_window(b):
    k0 = max(0, b * QBLK - WINDOW)
    k1 = min(SQ, (b + 1) * QBLK + WINDOW)
    return k0, k1 - k0


def _body(x_ref, wq_hbm, k_ref, v_ref, wo_hbm, out_ref,
          wq_sl, wo_sl, ctx_ref, comm_ref, copy_sems, send_sems, recv_sems):
    my = lax.axis_index("i")
    left = (my + N_DEV - 1) % N_DEV
    right = (my + 1) % N_DEV

    barrier = pltpu.get_barrier_semaphore()
    for nbr in (left, right):
        pl.semaphore_signal(barrier, inc=1, device_id=(nbr,),
                            device_id_type=pl.DeviceIdType.MESH)
    pl.semaphore_wait(barrier, 2)

    col0 = my * D_LOCAL
    cp_wq = pltpu.make_async_copy(
        wq_hbm.at[:, pl.ds(col0, D_LOCAL)], wq_sl, copy_sems.at[0])
    cp_wo = pltpu.make_async_copy(
        wo_hbm.at[pl.ds(col0, D_LOCAL), :], wo_sl, copy_sems.at[1])
    cp_wq.start()
    cp_wo.start()
    cp_wq.wait()

    xb = x_ref[...].astype(jnp.bfloat16)
    q = jnp.dot(xb, wq_sl[...].astype(jnp.bfloat16),
                preferred_element_type=jnp.float32)
    q = (q * SCALE).astype(jnp.bfloat16)

    for h in range(HQ_LOCAL):
        kh = k_ref[:, pl.ds(h * DH, DH)].astype(jnp.bfloat16)
        vh = v_ref[:, pl.ds(h * DH, DH)].astype(jnp.bfloat16)
        for b in range(N_QBLK):
            k0, wlen = _key_window(b)
            qb = q[b * QBLK:(b + 1) * QBLK, h * DH:(h + 1) * DH]
            kb = kh[k0:k0 + wlen]
            s = lax.dot_general(qb, kb, (((1,), (1,)), ((), ())),
                                preferred_element_type=jnp.float32)
            ri = lax.broadcasted_iota(jnp.int32, (QBLK, wlen), 0) + b * QBLK
            ci = lax.broadcasted_iota(jnp.int32, (QBLK, wlen), 1) + k0
            s = jnp.where(jnp.abs(ri - ci) <= WINDOW, s, -1e9)
            m = jnp.max(s, axis=1, keepdims=True)
            p = jnp.exp(s - m)
            p = (p / jnp.sum(p, axis=1, keepdims=True)).astype(jnp.bfloat16)
            ctx = jnp.dot(p, vh[k0:k0 + wlen],
                          preferred_element_type=jnp.float32)
            ctx_ref[b * QBLK:(b + 1) * QBLK, h * DH:(h + 1) * DH] = (
                ctx.astype(jnp.bfloat16))

    cp_wo.wait()
    partial = jnp.dot(ctx_ref[...], wo_sl[...].astype(jnp.bfloat16),
                      preferred_element_type=jnp.float32)
    out_ref[...] = partial
    comm_ref[0] = partial.astype(jnp.bfloat16)

    for hop in range(N_DEV - 1):
        rdma = pltpu.make_async_remote_copy(
            src_ref=comm_ref.at[hop],
            dst_ref=comm_ref.at[hop + 1],
            send_sem=send_sems.at[hop],
            recv_sem=recv_sems.at[hop],
            device_id=(right,),
            device_id_type=pl.DeviceIdType.MESH,
        )
        rdma.start()
        rdma.wait()
        out_ref[...] = out_ref[...] + comm_ref[hop + 1].astype(jnp.float32)


def kernel(x, Wq, K_ext, V_ext, Wo):
    x2 = x.reshape(SQ, D_MODEL)
    k2 = K_ext.reshape(SQ, HQ_LOCAL * DH)
    v2 = V_ext.reshape(SQ, HQ_LOCAL * DH)

    out = pl.pallas_call(
        _body,
        out_shape=jax.ShapeDtypeStruct((SQ, D_MODEL), jnp.float32),
        in_specs=[
            pl.BlockSpec(memory_space=pltpu.VMEM),
            pl.BlockSpec(memory_space=pltpu.ANY),
            pl.BlockSpec(memory_space=pltpu.VMEM),
            pl.BlockSpec(memory_space=pltpu.VMEM),
            pl.BlockSpec(memory_space=pltpu.ANY),
        ],
        out_specs=pl.BlockSpec(memory_space=pltpu.VMEM),
        scratch_shapes=[
            pltpu.VMEM((D_MODEL, D_LOCAL), jnp.float32),
            pltpu.VMEM((D_LOCAL, D_MODEL), jnp.float32),
            pltpu.VMEM((SQ, HQ_LOCAL * DH), jnp.bfloat16),
            pltpu.VMEM((N_DEV, SQ, D_MODEL), jnp.bfloat16),
            pltpu.SemaphoreType.DMA((2,)),
            pltpu.SemaphoreType.DMA((N_DEV - 1,)),
            pltpu.SemaphoreType.DMA((N_DEV - 1,)),
        ],
        compiler_params=pltpu.CompilerParams(collective_id=0),
    )(x2, Wq, k2, v2, Wo)
    return out.reshape(1, SQ, D_MODEL)


# baseline (device time: 221661 ns/iter reference)
import jax
import jax.numpy as jnp
from jax import lax
from jax.experimental import pallas as pl
from jax.experimental.pallas import tpu as pltpu

N_DEV = 4
SQ = 2048
D_MODEL = 1024
HQ_LOCAL = 8
DH = 128
WINDOW = 128
SCALE = 0.08838834764831843
QBLK = 512
N_QBLK = SQ // QBLK
D_LOCAL = HQ_LOCAL * DH


def _key_window(b):
    k0 = max(0, b * QBLK - WINDOW)
    k1 = min(SQ, (b + 1) * QBLK + WINDOW)
    return k0, k1 - k0


def _body(x_ref, wq_hbm, k_ref, v_ref, wo_hbm, out_ref,
          wq_sl, wo_sl, ctx_ref, comm_ref, copy_sems, send_sems, recv_sems):
    my = lax.axis_index("i")
    left = (my + N_DEV - 1) % N_DEV
    right = (my + 1) % N_DEV

    barrier = pltpu.get_barrier_semaphore()
    for nbr in (left, right):
        pl.semaphore_signal(barrier, inc=1, device_id=(nbr,),
                            device_id_type=pl.DeviceIdType.MESH)
    pl.semaphore_wait(barrier, 2)

    col0 = my * D_LOCAL
    cp_wq = pltpu.make_async_copy(
        wq_hbm.at[:, pl.ds(col0, D_LOCAL)], wq_sl, copy_sems.at[0])
    cp_wo = pltpu.make_async_copy(
        wo_hbm.at[pl.ds(col0, D_LOCAL), :], wo_sl, copy_sems.at[1])
    cp_wq.start()
    cp_wo.start()
    cp_wq.wait()

    q = jnp.dot(x_ref[...], wq_sl[...],
                preferred_element_type=jnp.float32)
    q = (q * SCALE).astype(jnp.bfloat16)

    for h in range(HQ_LOCAL):
        kh = k_ref[:, pl.ds(h * DH, DH)]
        vh = v_ref[:, pl.ds(h * DH, DH)]
        for b in range(N_QBLK):
            k0, wlen = _key_window(b)
            qb = q[b * QBLK:(b + 1) * QBLK, h * DH:(h + 1) * DH]
            kb = kh[k0:k0 + wlen]
            s = lax.dot_general(qb, kb, (((1,), (1,)), ((), ())),
                                preferred_element_type=jnp.float32)
            ri = lax.broadcasted_iota(jnp.int32, (QBLK, wlen), 0) + b * QBLK
            ci = lax.broadcasted_iota(jnp.int32, (QBLK, wlen), 1) + k0
            s = jnp.where(jnp.abs(ri - ci) <= WINDOW, s, -1e9)
            m = jnp.max(s, axis=1, keepdims=True)
            p = jnp.exp(s - m)
            p = (p / jnp.sum(p, axis=1, keepdims=True)).astype(jnp.bfloat16)
            ctx = jnp.dot(p, vh[k0:k0 + wlen],
                          preferred_element_type=jnp.float32)
            ctx_ref[b * QBLK:(b + 1) * QBLK, h * DH:(h + 1) * DH] = (
                ctx.astype(jnp.bfloat16))

    cp_wo.wait()
    partial = jnp.dot(ctx_ref[...], wo_sl[...],
                      preferred_element_type=jnp.float32)
    out_ref[...] = partial
    comm_ref[0] = partial.astype(jnp.bfloat16)

    for hop in range(N_DEV - 1):
        rdma = pltpu.make_async_remote_copy(
            src_ref=comm_ref.at[hop],
            dst_ref=comm_ref.at[hop + 1],
            send_sem=send_sems.at[hop],
            recv_sem=recv_sems.at[hop],
            device_id=(right,),
            device_id_type=pl.DeviceIdType.MESH,
        )
        rdma.start()
        rdma.wait()
        out_ref[...] = out_ref[...] + comm_ref[hop + 1].astype(jnp.float32)


def kernel(x, Wq, K_ext, V_ext, Wo):
    x2 = x.reshape(SQ, D_MODEL).astype(jnp.bfloat16)
    k2 = K_ext.reshape(SQ, HQ_LOCAL * DH).astype(jnp.bfloat16)
    v2 = V_ext.reshape(SQ, HQ_LOCAL * DH).astype(jnp.bfloat16)
    wq = Wq.astype(jnp.bfloat16)
    wo = Wo.astype(jnp.bfloat16)

    out = pl.pallas_call(
        _body,
        out_shape=jax.ShapeDtypeStruct((SQ, D_MODEL), jnp.float32),
        in_specs=[
            pl.BlockSpec(memory_space=pltpu.VMEM),
            pl.BlockSpec(memory_space=pl.ANY),
            pl.BlockSpec(memory_space=pltpu.VMEM),
            pl.BlockSpec(memory_space=pltpu.VMEM),
            pl.BlockSpec(memory_space=pl.ANY),
        ],
        out_specs=pl.BlockSpec(memory_space=pltpu.VMEM),
        scratch_shapes=[
            pltpu.VMEM((D_MODEL, D_LOCAL), jnp.bfloat16),
            pltpu.VMEM((D_LOCAL, D_MODEL), jnp.bfloat16),
            pltpu.VMEM((SQ, HQ_LOCAL * DH), jnp.bfloat16),
            pltpu.VMEM((N_DEV, SQ, D_MODEL), jnp.bfloat16),
            pltpu.SemaphoreType.DMA((2,)),
            pltpu.SemaphoreType.DMA((N_DEV - 1,)),
            pltpu.SemaphoreType.DMA((N_DEV - 1,)),
        ],
        compiler_params=pltpu.CompilerParams(
            collective_id=0, vmem_limit_bytes=100 * 1024 * 1024),
    )(x2, wq, k2, v2, wo)
    return out.reshape(1, SQ, D_MODEL)


# device time: 117274 ns/iter; 1.8901x vs baseline; 1.8901x over previous
import jax
import jax.numpy as jnp
from jax import lax
from jax.experimental import pallas as pl
from jax.experimental.pallas import tpu as pltpu

N_DEV = 4
SQ = 2048
D_MODEL = 1024
HQ_LOCAL = 8
DH = 128
WINDOW = 128
SCALE = 0.08838834764831843
QBLK = 512
KW = QBLK + 2 * WINDOW
D_LOCAL = HQ_LOCAL * DH


def _body(x_ref, wq_hbm, k_ref, v_ref, wo_hbm, out_ref,
          wq_sl, wo_sl, ctx_ref, sbuf, rbuf, gbuf,
          copy_sems, rs_send, rs_recv, ag_send, ag_recv):
    my = lax.axis_index("i")

    barrier = pltpu.get_barrier_semaphore()
    for rel in range(1, N_DEV):
        pl.semaphore_signal(barrier, inc=1, device_id=((my + rel) % N_DEV,),
                            device_id_type=pl.DeviceIdType.MESH)
    pl.semaphore_wait(barrier, N_DEV - 1)

    col0 = my * D_LOCAL
    cp_wq = pltpu.make_async_copy(
        wq_hbm.at[:, pl.ds(col0, D_LOCAL)], wq_sl, copy_sems.at[0])
    cp_wo = pltpu.make_async_copy(
        wo_hbm.at[pl.ds(col0, D_LOCAL), :], wo_sl, copy_sems.at[1])
    cp_wq.start()
    cp_wo.start()
    cp_wq.wait()
    cp_wo.wait()

    def compute_chunk(c):
        r0 = c * QBLK
        qb = jnp.dot(x_ref[pl.ds(r0, QBLK), :], wq_sl[...],
                     preferred_element_type=jnp.float32)
        qb = (qb * SCALE).astype(jnp.bfloat16)
        k0 = pl.multiple_of(jnp.clip(r0 - WINDOW, 0, SQ - KW), 128)
        kwin = k_ref[pl.ds(k0, KW), :]
        vwin = v_ref[pl.ds(k0, KW), :]
        for h in range(HQ_LOCAL):
            hs = slice(h * DH, (h + 1) * DH)
            s = lax.dot_general(qb[:, hs], kwin[:, hs],
                                (((1,), (1,)), ((), ())),
                                preferred_element_type=jnp.float32)
            ri = lax.broadcasted_iota(jnp.int32, (QBLK, KW), 0) + r0
            ci = lax.broadcasted_iota(jnp.int32, (QBLK, KW), 1) + k0
            s = jnp.where(jnp.abs(ri - ci) <= WINDOW, s, -1e9)
            m = jnp.max(s, axis=1, keepdims=True)
            p = jnp.exp(s - m)
            p = (p / jnp.sum(p, axis=1, keepdims=True)).astype(jnp.bfloat16)
            ctx = jnp.dot(p, vwin[:, hs],
                          preferred_element_type=jnp.float32)
            ctx_ref[:, hs] = ctx.astype(jnp.bfloat16)
        return jnp.dot(ctx_ref[...], wo_sl[...],
                       preferred_element_type=jnp.float32)

    rs_rdmas = []
    for s in range(N_DEV - 1):
        c = (my + 1 + s) % N_DEV
        part = compute_chunk(c)
        sbuf[s] = part.astype(jnp.bfloat16)
        rdma = pltpu.make_async_remote_copy(
            src_ref=sbuf.at[s],
            dst_ref=rbuf.at[2 - s],
            send_sem=rs_send.at[s],
            recv_sem=rs_recv.at[2 - s],
            device_id=(c,),
            device_id_type=pl.DeviceIdType.MESH,
        )
        rdma.start()
        rs_rdmas.append(rdma)

    acc = compute_chunk(my)
    for j in range(N_DEV - 1):
        recv = pltpu.make_async_remote_copy(
            src_ref=rbuf.at[j], dst_ref=rbuf.at[j],
            send_sem=rs_send.at[j], recv_sem=rs_recv.at[j],
            device_id=(my,), device_id_type=pl.DeviceIdType.MESH,
        )
        recv.wait_recv()
        acc = acc + rbuf[j].astype(jnp.float32)

    out_ref[pl.ds(my * QBLK, QBLK), :] = acc
    gbuf[pl.ds(my, 1), :, :] = acc.astype(jnp.bfloat16).reshape(1, QBLK, D_MODEL)

    ag_rdmas = []
    for rel in range(1, N_DEV):
        c = (my + rel) % N_DEV
        rdma = pltpu.make_async_remote_copy(
            src_ref=gbuf.at[my],
            dst_ref=gbuf.at[my],
            send_sem=ag_send.at[rel - 1],
            recv_sem=ag_recv.at[N_DEV - 1 - rel],
            device_id=(c,),
            device_id_type=pl.DeviceIdType.MESH,
        )
        rdma.start()
        ag_rdmas.append(rdma)

    for rel in range(1, N_DEV):
        c = (my + rel) % N_DEV
        recv = pltpu.make_async_remote_copy(
            src_ref=gbuf.at[rel - 1], dst_ref=gbuf.at[rel - 1],
            send_sem=ag_send.at[rel - 1], recv_sem=ag_recv.at[rel - 1],
            device_id=(my,), device_id_type=pl.DeviceIdType.MESH,
        )
        recv.wait_recv()

    for rel in range(1, N_DEV):
        c = (my + rel) % N_DEV
        chunk = gbuf[pl.ds(c, 1), :, :].reshape(QBLK, D_MODEL)
        out_ref[pl.ds(c * QBLK, QBLK), :] = chunk.astype(jnp.float32)

    for rdma in rs_rdmas + ag_rdmas:
        rdma.wait_send()


def kernel(x, Wq, K_ext, V_ext, Wo):
    x2 = x.reshape(SQ, D_MODEL).astype(jnp.bfloat16)
    k2 = K_ext.reshape(SQ, HQ_LOCAL * DH).astype(jnp.bfloat16)
    v2 = V_ext.reshape(SQ, HQ_LOCAL * DH).astype(jnp.bfloat16)
    wq = Wq.astype(jnp.bfloat16)
    wo = Wo.astype(jnp.bfloat16)

    out = pl.pallas_call(
        _body,
        out_shape=jax.ShapeDtypeStruct((SQ, D_MODEL), jnp.float32),
        in_specs=[
            pl.BlockSpec(memory_space=pltpu.VMEM),
            pl.BlockSpec(memory_space=pl.ANY),
            pl.BlockSpec(memory_space=pltpu.VMEM),
            pl.BlockSpec(memory_space=pltpu.VMEM),
            pl.BlockSpec(memory_space=pl.ANY),
        ],
        out_specs=pl.BlockSpec(memory_space=pltpu.VMEM),
        scratch_shapes=[
            pltpu.VMEM((D_MODEL, D_LOCAL), jnp.bfloat16),
            pltpu.VMEM((D_LOCAL, D_MODEL), jnp.bfloat16),
            pltpu.VMEM((QBLK, HQ_LOCAL * DH), jnp.bfloat16),
            pltpu.VMEM((N_DEV - 1, QBLK, D_MODEL), jnp.bfloat16),
            pltpu.VMEM((N_DEV - 1, QBLK, D_MODEL), jnp.bfloat16),
            pltpu.VMEM((N_DEV, QBLK, D_MODEL), jnp.bfloat16),
            pltpu.SemaphoreType.DMA((2,)),
            pltpu.SemaphoreType.DMA((N_DEV - 1,)),
            pltpu.SemaphoreType.DMA((N_DEV - 1,)),
            pltpu.SemaphoreType.DMA((N_DEV - 1,)),
            pltpu.SemaphoreType.DMA((N_DEV - 1,)),
        ],
        compiler_params=pltpu.CompilerParams(
            collective_id=0, vmem_limit_bytes=100 * 1024 * 1024),
    )(x2, wq, k2, v2, wo)
    return out.reshape(1, SQ, D_MODEL)


# device time: 101615 ns/iter; 2.1814x vs baseline; 1.1541x over previous
import jax
import jax.numpy as jnp
from jax import lax
from jax.experimental import pallas as pl
from jax.experimental.pallas import tpu as pltpu

N_DEV = 4
SQ = 2048
D_MODEL = 1024
HQ_LOCAL = 8
DH = 128
WINDOW = 128
SCALE = 0.08838834764831843
QBLK = 512
KW = QBLK + 2 * WINDOW
D_LOCAL = HQ_LOCAL * DH


def _k0_of(r0):
    return pl.multiple_of(jnp.clip(r0 - WINDOW, 0, SQ - KW), 128)


def _body(x_hbm, wq_hbm, k_hbm, v_hbm, wo_hbm, out_ref,
          wf, wq_b, wo_b, ctx_ref, xf, kf, vf, sbuf, rbuf, gbuf,
          w_sem, x_sems, k_sems, v_sems, rs_send, rs_recv, ag_send, ag_recv):
    my = lax.axis_index("i")

    barrier = pltpu.get_barrier_semaphore()
    for rel in range(1, N_DEV):
        pl.semaphore_signal(barrier, inc=1, device_id=((my + rel) % N_DEV,),
                            device_id_type=pl.DeviceIdType.MESH)
    pl.semaphore_wait(barrier, N_DEV - 1)

    def start_fetch(slot, c):
        r0 = c * QBLK
        k0 = _k0_of(r0)
        cps = (
            pltpu.make_async_copy(
                x_hbm.at[pl.ds(r0, QBLK), :], xf.at[slot], x_sems.at[slot]),
            pltpu.make_async_copy(
                k_hbm.at[pl.ds(k0, KW), :], kf.at[slot], k_sems.at[slot]),
            pltpu.make_async_copy(
                v_hbm.at[pl.ds(k0, KW), :], vf.at[slot], v_sems.at[slot]),
        )
        for cp in cps:
            cp.start()
        return cps

    fetches = {0: start_fetch(0, (my + 1) % N_DEV)}
    col0 = my * D_LOCAL
    cp_wq = pltpu.make_async_copy(
        wq_hbm.at[:, pl.ds(col0, D_LOCAL)], wf, w_sem)
    cp_wq.start()
    cp_wq.wait()
    wq_b[...] = (wf[...] * SCALE).astype(jnp.bfloat16)
    cp_wo = pltpu.make_async_copy(
        wo_hbm.at[pl.ds(col0, D_LOCAL), :], wf, w_sem)
    cp_wo.start()
    cp_wo.wait()
    wo_b[...] = wf[...].astype(jnp.bfloat16)

    def compute_chunk(s, c):
        slot = s % 2
        r0 = c * QBLK
        k0 = _k0_of(r0)
        for cp in fetches.pop(slot):
            cp.wait()
        if s < N_DEV - 1:
            fetches[(s + 1) % 2] = start_fetch((s + 1) % 2,
                                               (my + 2 + s) % N_DEV)
        qb = jnp.dot(xf[slot].astype(jnp.bfloat16), wq_b[...],
                     preferred_element_type=jnp.float32)
        qb = qb.astype(jnp.bfloat16)
        kwin = kf[slot].astype(jnp.bfloat16)
        vwin = vf[slot].astype(jnp.bfloat16)
        for h in range(HQ_LOCAL):
            hs = slice(h * DH, (h + 1) * DH)
            sc = lax.dot_general(qb[:, hs], kwin[:, hs],
                                 (((1,), (1,)), ((), ())),
                                 preferred_element_type=jnp.float32)
            ri = lax.broadcasted_iota(jnp.int32, (QBLK, KW), 0) + r0
            ci = lax.broadcasted_iota(jnp.int32, (QBLK, KW), 1) + k0
            sc = jnp.where(jnp.abs(ri - ci) <= WINDOW, sc, -1e9)
            m = jnp.max(sc, axis=1, keepdims=True)
            p = jnp.exp(sc - m)
            p = (p / jnp.sum(p, axis=1, keepdims=True)).astype(jnp.bfloat16)
            ctx = jnp.dot(p, vwin[:, hs],
                          preferred_element_type=jnp.float32)
            ctx_ref[:, hs] = ctx.astype(jnp.bfloat16)
        return jnp.dot(ctx_ref[...], wo_b[...],
                       preferred_element_type=jnp.float32)

    rs_rdmas = []
    for s in range(N_DEV - 1):
        c = (my + 1 + s) % N_DEV
        part = compute_chunk(s, c)
        sbuf[s] = part.astype(jnp.bfloat16)
        rdma = pltpu.make_async_remote_copy(
            src_ref=sbuf.at[s],
            dst_ref=rbuf.at[2 - s],
            send_sem=rs_send.at[s],
            recv_sem=rs_recv.at[2 - s],
            device_id=(c,),
            device_id_type=pl.DeviceIdType.MESH,
        )
        rdma.start()
        rs_rdmas.append(rdma)

    acc = compute_chunk(N_DEV - 1, my)
    for j in range(N_DEV - 1):
        recv = pltpu.make_async_remote_copy(
            src_ref=rbuf.at[j], dst_ref=rbuf.at[j],
            send_sem=rs_send.at[j], recv_sem=rs_recv.at[j],
            device_id=(my,), device_id_type=pl.DeviceIdType.MESH,
        )
        recv.wait_recv()
        acc = acc + rbuf[j].astype(jnp.float32)

    out_ref[pl.ds(my * QBLK, QBLK), :] = acc
    gbuf[pl.ds(my, 1), :, :] = acc.astype(jnp.bfloat16).reshape(
        1, QBLK, D_MODEL)

    ag_rdmas = []
    for rel in range(1, N_DEV):
        c = (my + rel) % N_DEV
        rdma = pltpu.make_async_remote_copy(
            src_ref=gbuf.at[my],
            dst_ref=gbuf.at[my],
            send_sem=ag_send.at[rel - 1],
            recv_sem=ag_recv.at[N_DEV - 1 - rel],
            device_id=(c,),
            device_id_type=pl.DeviceIdType.MESH,
        )
        rdma.start()
        ag_rdmas.append(rdma)

    for rel in range(1, N_DEV):
        recv = pltpu.make_async_remote_copy(
            src_ref=gbuf.at[rel - 1], dst_ref=gbuf.at[rel - 1],
            send_sem=ag_send.at[rel - 1], recv_sem=ag_recv.at[rel - 1],
            device_id=(my,), device_id_type=pl.DeviceIdType.MESH,
        )
        recv.wait_recv()

    for rel in range(1, N_DEV):
        c = (my + rel) % N_DEV
        chunk = gbuf[pl.ds(c, 1), :, :].reshape(QBLK, D_MODEL)
        out_ref[pl.ds(c * QBLK, QBLK), :] = chunk.astype(jnp.float32)

    for rdma in rs_rdmas + ag_rdmas:
        rdma.wait_send()


def kernel(x, Wq, K_ext, V_ext, Wo):
    x2 = x.reshape(SQ, D_MODEL)
    k2 = K_ext.reshape(SQ, HQ_LOCAL * DH)
    v2 = V_ext.reshape(SQ, HQ_LOCAL * DH)

    out = pl.pallas_call(
        _body,
        out_shape=jax.ShapeDtypeStruct((SQ, D_MODEL), jnp.float32),
        in_specs=[pl.BlockSpec(memory_space=pl.ANY)] * 5,
        out_specs=pl.BlockSpec(memory_space=pltpu.VMEM),
        scratch_shapes=[
            pltpu.VMEM((D_MODEL, D_LOCAL), jnp.float32),
            pltpu.VMEM((D_MODEL, D_LOCAL), jnp.bfloat16),
            pltpu.VMEM((D_LOCAL, D_MODEL), jnp.bfloat16),
            pltpu.VMEM((QBLK, HQ_LOCAL * DH), jnp.bfloat16),
            pltpu.VMEM((2, QBLK, D_MODEL), jnp.float32),
            pltpu.VMEM((2, KW, HQ_LOCAL * DH), jnp.float32),
            pltpu.VMEM((2, KW, HQ_LOCAL * DH), jnp.float32),
            pltpu.VMEM((N_DEV - 1, QBLK, D_MODEL), jnp.bfloat16),
            pltpu.VMEM((N_DEV - 1, QBLK, D_MODEL), jnp.bfloat16),
            pltpu.VMEM((N_DEV, QBLK, D_MODEL), jnp.bfloat16),
            pltpu.SemaphoreType.DMA,
            pltpu.SemaphoreType.DMA((2,)),
            pltpu.SemaphoreType.DMA((2,)),
            pltpu.SemaphoreType.DMA((2,)),
            pltpu.SemaphoreType.DMA((N_DEV - 1,)),
            pltpu.SemaphoreType.DMA((N_DEV - 1,)),
            pltpu.SemaphoreType.DMA((N_DEV - 1,)),
            pltpu.SemaphoreType.DMA((N_DEV - 1,)),
        ],
        compiler_params=pltpu.CompilerParams(
            collective_id=0, vmem_limit_bytes=100 * 1024 * 1024),
    )(x2, Wq, k2, v2, Wo)
    return out.reshape(1, SQ, D_MODEL)


# device time: 84789 ns/iter; 2.6143x vs baseline; 1.1984x over previous
import jax
import jax.numpy as jnp
from jax import lax
from jax.experimental import pallas as pl
from jax.experimental.pallas import tpu as pltpu

N_DEV = 4
SQ = 2048
D_MODEL = 1024
HQ_LOCAL = 8
DH = 128
WINDOW = 128
SCALE = 0.08838834764831843
QBLK = 512
KW = QBLK + 2 * WINDOW
D_LOCAL = HQ_LOCAL * DH


def _k0_of(r0):
    return pl.multiple_of(jnp.clip(r0 - WINDOW, 0, SQ - KW), 128)


def _body(x_hbm, wq_hbm, k_hbm, v_hbm, wo_hbm, out_ref,
          wf, wq_b, wo_b, ctx_ref, xf, kf, vf, sbuf, rbuf, gbuf,
          w_sem, x_sems, k_sems, v_sems, rs_send, rs_recv, ag_send, ag_recv):
    my = lax.axis_index("i")

    barrier = pltpu.get_barrier_semaphore()
    for rel in range(1, N_DEV):
        pl.semaphore_signal(barrier, inc=1, device_id=((my + rel) % N_DEV,),
                            device_id_type=pl.DeviceIdType.MESH)
    pl.semaphore_wait(barrier, N_DEV - 1)

    def start_fetch(slot, c):
        r0 = c * QBLK
        k0 = _k0_of(r0)
        cps = [pltpu.make_async_copy(
            x_hbm.at[0, pl.ds(r0, QBLK), :], xf.at[slot], x_sems.at[slot])]
        for h in range(HQ_LOCAL):
            cps.append(pltpu.make_async_copy(
                k_hbm.at[0, pl.ds(k0, KW), h, :], kf.at[slot, h],
                k_sems.at[slot, h]))
            cps.append(pltpu.make_async_copy(
                v_hbm.at[0, pl.ds(k0, KW), h, :], vf.at[slot, h],
                v_sems.at[slot, h]))
        for cp in cps:
            cp.start()
        return cps

    fetches = {0: start_fetch(0, (my + 1) % N_DEV)}
    col0 = my * D_LOCAL
    cp_wq = pltpu.make_async_copy(
        wq_hbm.at[:, pl.ds(col0, D_LOCAL)], wf, w_sem)
    cp_wq.start()
    cp_wq.wait()
    wq_b[...] = (wf[...] * SCALE).astype(jnp.bfloat16)
    cp_wo = pltpu.make_async_copy(
        wo_hbm.at[pl.ds(col0, D_LOCAL), :], wf, w_sem)
    cp_wo.start()
    cp_wo.wait()
    wo_b[...] = wf[...].astype(jnp.bfloat16)

    def compute_chunk(s, c):
        slot = s % 2
        r0 = c * QBLK
        k0 = _k0_of(r0)
        for cp in fetches.pop(slot):
            cp.wait()
        if s < N_DEV - 1:
            fetches[(s + 1) % 2] = start_fetch((s + 1) % 2,
                                               (my + 2 + s) % N_DEV)
        qb = jnp.dot(xf[slot].astype(jnp.bfloat16), wq_b[...],
                     preferred_element_type=jnp.float32)
        qb = qb.astype(jnp.bfloat16)
        ri = lax.broadcasted_iota(jnp.int32, (QBLK, KW), 0) + r0
        ci = lax.broadcasted_iota(jnp.int32, (QBLK, KW), 1) + k0
        maskf = (jnp.abs(ri - ci) <= WINDOW).astype(jnp.float32)
        for h in range(HQ_LOCAL):
            hs = slice(h * DH, (h + 1) * DH)
            kh = kf[slot, h].astype(jnp.bfloat16)
            vh = vf[slot, h].astype(jnp.bfloat16)
            sc = lax.dot_general(qb[:, hs], kh,
                                 (((1,), (1,)), ((), ())),
                                 preferred_element_type=jnp.float32)
            p = jnp.exp(sc) * maskf
            p = (p / jnp.sum(p, axis=1, keepdims=True)).astype(jnp.bfloat16)
            ctx = jnp.dot(p, vh, preferred_element_type=jnp.float32)
            ctx_ref[:, hs] = ctx.astype(jnp.bfloat16)
        return jnp.dot(ctx_ref[...], wo_b[...],
                       preferred_element_type=jnp.float32)

    rs_rdmas = []
    for s in range(N_DEV - 1):
        c = (my + 1 + s) % N_DEV
        part = compute_chunk(s, c)
        sbuf[s] = part.astype(jnp.bfloat16)
        rdma = pltpu.make_async_remote_copy(
            src_ref=sbuf.at[s],
            dst_ref=rbuf.at[2 - s],
            send_sem=rs_send.at[s],
            recv_sem=rs_recv.at[2 - s],
            device_id=(c,),
            device_id_type=pl.DeviceIdType.MESH,
        )
        rdma.start()
        rs_rdmas.append(rdma)

    acc = compute_chunk(N_DEV - 1, my)
    for j in range(N_DEV - 1):
        recv = pltpu.make_async_remote_copy(
            src_ref=rbuf.at[j], dst_ref=rbuf.at[j],
            send_sem=rs_send.at[j], recv_sem=rs_recv.at[j],
            device_id=(my,), device_id_type=pl.DeviceIdType.MESH,
        )
        recv.wait_recv()
        acc = acc + rbuf[j].astype(jnp.float32)

    out_ref[0, pl.ds(my * QBLK, QBLK), :] = acc
    gbuf[pl.ds(my, 1), :, :] = acc.astype(jnp.bfloat16).reshape(
        1, QBLK, D_MODEL)

    ag_rdmas = []
    for rel in range(1, N_DEV):
        c = (my + rel) % N_DEV
        rdma = pltpu.make_async_remote_copy(
            src_ref=gbuf.at[my],
            dst_ref=gbuf.at[my],
            send_sem=ag_send.at[rel - 1],
            recv_sem=ag_recv.at[N_DEV - 1 - rel],
            device_id=(c,),
            device_id_type=pl.DeviceIdType.MESH,
        )
        rdma.start()
        ag_rdmas.append(rdma)

    for rel in range(1, N_DEV):
        recv = pltpu.make_async_remote_copy(
            src_ref=gbuf.at[rel - 1], dst_ref=gbuf.at[rel - 1],
            send_sem=ag_send.at[rel - 1], recv_sem=ag_recv.at[rel - 1],
            device_id=(my,), device_id_type=pl.DeviceIdType.MESH,
        )
        recv.wait_recv()

    for rel in range(1, N_DEV):
        c = (my + rel) % N_DEV
        chunk = gbuf[pl.ds(c, 1), :, :].reshape(QBLK, D_MODEL)
        out_ref[0, pl.ds(c * QBLK, QBLK), :] = chunk.astype(jnp.float32)

    for rdma in rs_rdmas + ag_rdmas:
        rdma.wait_send()


def kernel(x, Wq, K_ext, V_ext, Wo):
    out = pl.pallas_call(
        _body,
        out_shape=jax.ShapeDtypeStruct((1, SQ, D_MODEL), jnp.float32),
        in_specs=[pl.BlockSpec(memory_space=pl.ANY)] * 5,
        out_specs=pl.BlockSpec(memory_space=pltpu.VMEM),
        scratch_shapes=[
            pltpu.VMEM((D_MODEL, D_LOCAL), jnp.float32),
            pltpu.VMEM((D_MODEL, D_LOCAL), jnp.bfloat16),
            pltpu.VMEM((D_LOCAL, D_MODEL), jnp.bfloat16),
            pltpu.VMEM((QBLK, HQ_LOCAL * DH), jnp.bfloat16),
            pltpu.VMEM((2, QBLK, D_MODEL), jnp.float32),
            pltpu.VMEM((2, HQ_LOCAL, KW, DH), jnp.float32),
            pltpu.VMEM((2, HQ_LOCAL, KW, DH), jnp.float32),
            pltpu.VMEM((N_DEV - 1, QBLK, D_MODEL), jnp.bfloat16),
            pltpu.VMEM((N_DEV - 1, QBLK, D_MODEL), jnp.bfloat16),
            pltpu.VMEM((N_DEV, QBLK, D_MODEL), jnp.bfloat16),
            pltpu.SemaphoreType.DMA,
            pltpu.SemaphoreType.DMA((2,)),
            pltpu.SemaphoreType.DMA((2, HQ_LOCAL)),
            pltpu.SemaphoreType.DMA((2, HQ_LOCAL)),
            pltpu.SemaphoreType.DMA((N_DEV - 1,)),
            pltpu.SemaphoreType.DMA((N_DEV - 1,)),
            pltpu.SemaphoreType.DMA((N_DEV - 1,)),
            pltpu.SemaphoreType.DMA((N_DEV - 1,)),
        ],
        compiler_params=pltpu.CompilerParams(
            collective_id=0, vmem_limit_bytes=100 * 1024 * 1024),
    )(x, Wq, K_ext, V_ext, Wo)
    return out
